# baseline (speedup 1.0000x reference)
import jax
import jax.numpy as jnp
from jax.experimental import pallas as pl
from jax.experimental.pallas import tpu as pltpu

_TOKENS = 16384
_N_EMBED = 4096
_N_EXP = 64
_K = 8
_BLK_T = 1024

_consts = {}


def _gauss():
    if "g" not in _consts:
        _consts["g"] = jax.random.normal(
            jax.random.key(42), (_TOKENS, _N_EXP), dtype=jnp.float32)
    return _consts["g"]


def _router_kernel(x_ref, w_ref, b_ref, g_ref, out_ref, idx_ref):
    x = x_ref[...].astype(jnp.bfloat16)
    w = w_ref[...].astype(jnp.bfloat16)
    acc = jax.lax.dot_general(
        x, w, (((1,), (0,)), ((), ())), preferred_element_type=jnp.float32)
    acc = acc + b_ref[...]
    logits = acc[:, :_N_EXP]
    nlog = acc[:, _N_EXP:]
    noisy = logits + g_ref[...] * jax.nn.softplus(nlog)

    i = jax.lax.bitcast_convert_type(noisy, jnp.int32)
    key = jnp.where(i < 0, i ^ jnp.int32(0x7FFFFFFF), i)
    lane = jax.lax.broadcasted_iota(jnp.int32, noisy.shape, 1)
    packed = (key & jnp.int32(-64)) | (jnp.int32(63) - lane)

    li = jax.lax.broadcasted_iota(jnp.int32, (1, _N_EXP), 1)
    s = packed
    k = 2
    while k <= _N_EXP:
        j = k // 2
        while j >= 1:
            bitj0 = (li & j) == 0
            take_max = ((li & k) == 0) == bitj0
            partner = jnp.where(bitj0,
                                pltpu.roll(s, _N_EXP - j, axis=1),
                                pltpu.roll(s, j, axis=1))
            s = jnp.where(take_max,
                          jnp.maximum(s, partner),
                          jnp.minimum(s, partner))
            j //= 2
        k *= 2

    idx_ref[...] = jnp.int32(63) - (s[:, :_K] & jnp.int32(63))

    t8 = s[:, _K - 1:_K]
    mask = packed >= t8
    bk = s[:, 0:1] | jnp.int32(63)
    bbits = jnp.where(bk < 0, bk ^ jnp.int32(0x7FFFFFFF), bk)
    bound = jax.lax.bitcast_convert_type(bbits, jnp.float32)
    e = jnp.where(mask, jnp.exp(noisy - bound), 0.0)
    out_ref[...] = e / jnp.sum(e, axis=-1, keepdims=True)


def kernel(mh_output, W_route, b_route, W_noise, b_noise):
    w_cat = jnp.concatenate([W_route, W_noise], axis=1)
    b_cat = jnp.concatenate([b_route, b_noise])[None, :]
    grid = (_TOKENS // _BLK_T,)
    router, indices = pl.pallas_call(
        _router_kernel,
        grid=grid,
        in_specs=[
            pl.BlockSpec((_BLK_T, _N_EMBED), lambda t: (t, 0)),
            pl.BlockSpec((_N_EMBED, 2 * _N_EXP), lambda t: (0, 0)),
            pl.BlockSpec((1, 2 * _N_EXP), lambda t: (0, 0)),
            pl.BlockSpec((_BLK_T, _N_EXP), lambda t: (t, 0)),
        ],
        out_specs=[
            pl.BlockSpec((_BLK_T, _N_EXP), lambda t: (t, 0)),
            pl.BlockSpec((_BLK_T, _K), lambda t: (t, 0)),
        ],
        out_shape=[
            jax.ShapeDtypeStruct((_TOKENS, _N_EXP), jnp.float32),
            jax.ShapeDtypeStruct((_TOKENS, _K), jnp.int32),
        ],
        compiler_params=pltpu.CompilerParams(
            dimension_semantics=("parallel",)),
    )(mh_output, w_cat, b_cat, _gauss())
    return (router, indices)


# speedup vs baseline: 2.2472x; 2.2472x over previous
import jax
import jax.numpy as jnp
from jax.experimental import pallas as pl
from jax.experimental.pallas import tpu as pltpu

_TOKENS = 16384
_N_EMBED = 4096
_N_EXP = 64
_K = 8
_BLK_T = 1024

_consts = {}


def _gauss_t():
    if "g" not in _consts:
        g = jax.random.normal(
            jax.random.key(42), (_TOKENS, _N_EXP), dtype=jnp.float32)
        _consts["g"] = jnp.transpose(g)
    return _consts["g"]


def _router_kernel(x_ref, w_ref, b_ref, g_ref, out_ref, idx_ref):
    x = x_ref[...].astype(jnp.bfloat16)
    w = w_ref[...].astype(jnp.bfloat16)
    acc = jax.lax.dot_general(
        x, w, (((1,), (0,)), ((), ())), preferred_element_type=jnp.float32)
    acc = acc + b_ref[...]
    acc_t = jnp.transpose(acc)
    logits = acc_t[:_N_EXP, :]
    nlog = acc_t[_N_EXP:, :]
    noisy = logits + g_ref[...] * jax.nn.softplus(nlog)

    eidx = jax.lax.broadcasted_iota(jnp.int32, (_N_EXP, _BLK_T), 0)
    slot = jax.lax.broadcasted_iota(jnp.int32, (_K, _BLK_T), 0)
    work = noisy
    mask = jnp.zeros(noisy.shape, jnp.bool_)
    idxs_t = jnp.zeros((_K, _BLK_T), jnp.int32)
    vmax = None
    for j in range(_K):
        m = jnp.max(work, axis=0, keepdims=True)
        if j == 0:
            vmax = m
        sel = work == m
        win = jnp.min(jnp.where(sel, eidx, _N_EXP), axis=0, keepdims=True)
        idxs_t = jnp.where(slot == j, win, idxs_t)
        chosen = jnp.logical_and(sel, eidx == win)
        mask = jnp.logical_or(mask, chosen)
        work = jnp.where(chosen, -jnp.inf, work)
    idx_ref[...] = jnp.transpose(idxs_t)

    e = jnp.where(mask, jnp.exp(noisy - vmax), 0.0)
    sm = e / jnp.sum(e, axis=0, keepdims=True)
    out_ref[...] = jnp.transpose(sm)


def kernel(mh_output, W_route, b_route, W_noise, b_noise):
    w_cat = jnp.concatenate([W_route, W_noise], axis=1)
    b_cat = jnp.concatenate([b_route, b_noise])[None, :]
    grid = (_TOKENS // _BLK_T,)
    router, indices = pl.pallas_call(
        _router_kernel,
        grid=grid,
        in_specs=[
            pl.BlockSpec((_BLK_T, _N_EMBED), lambda t: (t, 0)),
            pl.BlockSpec((_N_EMBED, 2 * _N_EXP), lambda t: (0, 0)),
            pl.BlockSpec((1, 2 * _N_EXP), lambda t: (0, 0)),
            pl.BlockSpec((_N_EXP, _BLK_T), lambda t: (0, t)),
        ],
        out_specs=[
            pl.BlockSpec((_BLK_T, _N_EXP), lambda t: (t, 0)),
            pl.BlockSpec((_BLK_T, _K), lambda t: (t, 0)),
        ],
        out_shape=[
            jax.ShapeDtypeStruct((_TOKENS, _N_EXP), jnp.float32),
            jax.ShapeDtypeStruct((_TOKENS, _K), jnp.int32),
        ],
        compiler_params=pltpu.CompilerParams(
            dimension_semantics=("parallel",)),
    )(mh_output, w_cat, b_cat, _gauss_t())
    return (router, indices)
